# baseline (speedup 1.0000x reference)
import functools

import jax
import jax.numpy as jnp
from jax import lax
from jax.experimental import pallas as pl
from jax.experimental.pallas import tpu as pltpu
from jax.experimental.pallas import tpu_sc as plsc


_BB = 256


def _argmin_body(q_ref, ktT_ref, t2_ref, q2_ref, idx_ref, d2s_ref, iota_ref):
    @pl.when(pl.program_id(0) == 0)
    def _():
        iota_ref[...] = lax.broadcasted_iota(jnp.int32, iota_ref.shape, 1)

    q2 = q2_ref[...]
    qt = jnp.dot(q_ref[...], ktT_ref[...], preferred_element_type=jnp.float32)
    d2 = (q2 + t2_ref[...]) - 2.0 * qt
    d2s_ref[...] = d2
    m2 = jnp.min(d2, axis=1, keepdims=True)
    s = jnp.sqrt(jnp.maximum(m2, 0.0))
    yb = lax.bitcast_convert_type(s * s, jnp.int32)
    hi = jnp.full_like(s, -jnp.inf)
    for koff in range(-4, 5):
        xk = lax.bitcast_convert_type(jnp.maximum(yb + koff, 0), jnp.float32)
        ok = jnp.sqrt(jnp.maximum(xk, 0.0)) == s
        hi = jnp.maximum(hi, jnp.where(ok, xk, -jnp.inf))
    hi = jnp.where(s == 0.0, 0.0, hi)
    k = d2.shape[1]
    cand = jnp.where(d2s_ref[...] <= hi, iota_ref[...], k)
    idx_ref[...] = jnp.min(cand, axis=1).astype(jnp.int32)


def _nearest_idx(q, ktT, t2, q2):
    b, dk = q.shape
    k = ktT.shape[1]
    return pl.pallas_call(
        _argmin_body,
        grid=(b // _BB,),
        in_specs=[
            pl.BlockSpec((_BB, dk), lambda i: (i, 0)),
            pl.BlockSpec((dk, k), lambda i: (0, 0)),
            pl.BlockSpec((1, k), lambda i: (0, 0)),
            pl.BlockSpec((_BB, 1), lambda i: (i, 0)),
        ],
        out_specs=pl.BlockSpec((_BB,), lambda i: (i,)),
        out_shape=jax.ShapeDtypeStruct((b,), jnp.int32),
        scratch_shapes=[
            pltpu.VMEM((_BB, k), jnp.float32),
            pltpu.VMEM((_BB, k), jnp.int32),
        ],
    )(q, ktT, t2, q2)


_NC, _NS, _L = 2, 16, 16
_NW = _NC * _NS


def _sc_linearize(table):
    kk, c_total = table.shape
    ntr = kk // 8
    trw = ntr // _NW
    mesh = plsc.VectorSubcoreMesh(
        core_axis_name="c", subcore_axis_name="s",
        num_cores=_NC, num_subcores=_NS)

    @functools.partial(
        pl.kernel,
        out_type=jax.ShapeDtypeStruct((ntr * 2 * 8, 128), jnp.float32),
        mesh=mesh,
        scratch_types=[
            pltpu.VMEM((trw * 2 * 8, 128), jnp.float32),
            pltpu.SemaphoreType.DMA,
        ],
        compiler_params=pltpu.CompilerParams(
            use_tc_tiling_on_sc=True, needs_layout_passes=False),
    )
    def body(tbl_hbm, out_hbm, vbuf, sem):
        wid = lax.axis_index("s") * _NC + lax.axis_index("c")
        tr0 = wid * trw
        cps = []
        for t in range(trw):
            for ct in range(2):
                cps.append(pltpu.async_copy(
                    tbl_hbm.at[pl.ds((tr0 + t) * 8, 8), pl.ds(ct * 128, 128)],
                    vbuf.at[pl.ds((t * 2 + ct) * 8, 8)],
                    sem))
        for cp in cps:
            cp.wait()
        pltpu.sync_copy(vbuf, out_hbm.at[pl.ds(wid * trw * 2 * 8, trw * 2 * 8)])

    return body(table)


def _sc_residual(lin, tail, idx, feat, fidx):
    b = idx.shape[0]
    df = fidx.shape[0]
    tail_w = tail.shape[1]
    bpw = b // _NW
    nch = df // _L
    mesh = plsc.VectorSubcoreMesh(
        core_axis_name="c", subcore_axis_name="s",
        num_cores=_NC, num_subcores=_NS)

    @functools.partial(
        pl.kernel,
        out_type=jax.ShapeDtypeStruct((b, df), jnp.float32),
        mesh=mesh,
        scratch_types=[
            pltpu.VMEM((bpw,), jnp.int32),
            pltpu.VMEM((2 * bpw,), jnp.int32),
            pltpu.VMEM((2 * bpw, 128), jnp.float32),
            pltpu.VMEM((bpw, tail.shape[1]), jnp.float32),
            pltpu.VMEM((bpw, df), jnp.float32),
            pltpu.VMEM((df,), jnp.int32),
            pltpu.SemaphoreType.DMA,
            pltpu.SemaphoreType.DMA,
        ],
        compiler_params=pltpu.CompilerParams(
            use_tc_tiling_on_sc=False, needs_layout_passes=False),
    )
    def body(lin_hbm, tail_hbm, idx_hbm, feat_hbm, fidx_hbm, out_hbm,
             idx_v, idx2_v, rows_v, tailrows_v, feat_v, fidx_v, sem, sem2):
        wid = lax.axis_index("s") * _NC + lax.axis_index("c")
        base = wid * bpw
        pltpu.sync_copy(idx_hbm.at[pl.ds(base, bpw)], idx_v)
        cp2 = pltpu.async_copy(tail_hbm.at[idx_v], tailrows_v, sem2)
        lane = lax.iota(jnp.int32, _L)
        for g in range(bpw // _L):
            v = idx_v[pl.ds(g * _L, _L)]
            b2 = ((v >> 3) << 4) | (v & 7)
            pos = 2 * lane + (2 * g * _L)
            plsc.store_scatter(idx2_v, [pos], b2)
            plsc.store_scatter(idx2_v, [pos + 1], b2 + 8)
        cp = pltpu.async_copy(lin_hbm.at[idx2_v], rows_v, sem)
        pltpu.sync_copy(feat_hbm.at[pl.ds(base, bpw)], feat_v)
        pltpu.sync_copy(fidx_hbm, fidx_v)
        cp2.wait()
        cp.wait()
        csel, clane, ctail, cmask = [], [], [], []
        for c in range(nch):
            cols = fidx_v[pl.ds(c * _L, _L)]
            csel.append(jnp.minimum(cols >> 7, 1))
            clane.append(cols & 127)
            ctail.append(jnp.clip(cols - 2 * 128, 0, tail_w - 1))
            cmask.append(cols >= 2 * 128)

        def row_body(r, carry):
            r2 = jnp.full((_L,), 2 * r, jnp.int32)
            rs = jnp.full((_L,), r, jnp.int32)
            for c in range(nch):
                va = plsc.load_gather(rows_v, [r2 + csel[c], clane[c]])
                vb = plsc.load_gather(tailrows_v, [rs, ctail[c]])
                vals = jnp.where(cmask[c], vb, va)
                feat_v[r, pl.ds(c * _L, _L)] = feat_v[r, pl.ds(c * _L, _L)] - vals
            return carry

        lax.fori_loop(0, bpw, row_body, 0)
        pltpu.sync_copy(feat_v, out_hbm.at[pl.ds(base, bpw)])

    return body(lin, tail, idx, feat, fidx)


_NCHUNKS = 2


def kernel(predicted_key, features, lookup_table, lookup_key_indices,
           feature_indices):
    b = predicted_key.shape[0]
    key_table = jnp.take(lookup_table, lookup_key_indices, axis=1)
    t2 = jnp.sum(key_table * key_table, axis=1)[None, :]
    q2 = jnp.sum(predicted_key * predicted_key, axis=1, keepdims=True)
    ktT = key_table.T
    lin = _sc_linearize(lookup_table)
    tail = lax.slice(lookup_table, (0, 256), lookup_table.shape)
    bc = b // _NCHUNKS
    outs = []
    for c in range(_NCHUNKS):
        sl = slice(c * bc, (c + 1) * bc)
        idx_c = _nearest_idx(predicted_key[sl], ktT, t2, q2[sl])
        outs.append(_sc_residual(lin, tail, idx_c, features[sl],
                                 feature_indices))
    return jnp.concatenate(outs, axis=0)


# speedup vs baseline: 1.0519x; 1.0519x over previous
import functools

import jax
import jax.numpy as jnp
from jax import lax
from jax.experimental import pallas as pl
from jax.experimental.pallas import tpu as pltpu
from jax.experimental.pallas import tpu_sc as plsc


_BB = 256


def _argmin_body(q_ref, ktT_ref, t2_ref, q2_ref, idx_ref, d2s_ref, iota_ref):
    @pl.when(pl.program_id(0) == 0)
    def _():
        iota_ref[...] = lax.broadcasted_iota(jnp.int32, iota_ref.shape, 1)

    q2 = q2_ref[...]
    qt = jnp.dot(q_ref[...], ktT_ref[...], preferred_element_type=jnp.float32)
    d2 = (q2 + t2_ref[...]) - 2.0 * qt
    d2s_ref[...] = d2
    m2 = jnp.min(d2, axis=1, keepdims=True)
    s = jnp.sqrt(jnp.maximum(m2, 0.0))
    yb = lax.bitcast_convert_type(s * s, jnp.int32)
    hi = jnp.full_like(s, -jnp.inf)
    for koff in range(-4, 5):
        xk = lax.bitcast_convert_type(jnp.maximum(yb + koff, 0), jnp.float32)
        ok = jnp.sqrt(jnp.maximum(xk, 0.0)) == s
        hi = jnp.maximum(hi, jnp.where(ok, xk, -jnp.inf))
    hi = jnp.where(s == 0.0, 0.0, hi)
    k = d2.shape[1]
    cand = jnp.where(d2s_ref[...] <= hi, iota_ref[...], k)
    idx_ref[...] = jnp.min(cand, axis=1).astype(jnp.int32)


def _nearest_idx(q, ktT, t2, q2, chunk, bc):
    _, dk = q.shape
    k = ktT.shape[1]
    nsteps = bc // _BB
    off = chunk * nsteps
    return pl.pallas_call(
        _argmin_body,
        grid=(nsteps,),
        in_specs=[
            pl.BlockSpec((_BB, dk), lambda i: (off + i, 0)),
            pl.BlockSpec((dk, k), lambda i: (0, 0)),
            pl.BlockSpec((1, k), lambda i: (0, 0)),
            pl.BlockSpec((_BB, 1), lambda i: (off + i, 0)),
        ],
        out_specs=pl.BlockSpec((_BB,), lambda i: (i,)),
        out_shape=jax.ShapeDtypeStruct((bc,), jnp.int32),
        scratch_shapes=[
            pltpu.VMEM((_BB, k), jnp.float32),
            pltpu.VMEM((_BB, k), jnp.int32),
        ],
    )(q, ktT, t2, q2)


_NC, _NS, _L = 2, 16, 16
_NW = _NC * _NS


def _sc_linearize(table):
    kk, c_total = table.shape
    ntr = kk // 8
    trw = ntr // _NW
    mesh = plsc.VectorSubcoreMesh(
        core_axis_name="c", subcore_axis_name="s",
        num_cores=_NC, num_subcores=_NS)

    @functools.partial(
        pl.kernel,
        out_type=jax.ShapeDtypeStruct((ntr * 2 * 8, 128), jnp.float32),
        mesh=mesh,
        scratch_types=[
            pltpu.VMEM((trw * 2 * 8, 128), jnp.float32),
            pltpu.SemaphoreType.DMA,
        ],
        compiler_params=pltpu.CompilerParams(
            use_tc_tiling_on_sc=True, needs_layout_passes=False),
    )
    def body(tbl_hbm, out_hbm, vbuf, sem):
        wid = lax.axis_index("s") * _NC + lax.axis_index("c")
        tr0 = wid * trw
        cps = []
        for t in range(trw):
            for ct in range(2):
                cps.append(pltpu.async_copy(
                    tbl_hbm.at[pl.ds((tr0 + t) * 8, 8), pl.ds(ct * 128, 128)],
                    vbuf.at[pl.ds((t * 2 + ct) * 8, 8)],
                    sem))
        for cp in cps:
            cp.wait()
        pltpu.sync_copy(vbuf, out_hbm.at[pl.ds(wid * trw * 2 * 8, trw * 2 * 8)])

    return body(table)


def _sc_residual(lin, tail, idx, feat, fidx, chunk):
    b = idx.shape[0]
    df = fidx.shape[0]
    tail_w = tail.shape[1]
    bpw = b // _NW
    nch = df // _L
    mesh = plsc.VectorSubcoreMesh(
        core_axis_name="c", subcore_axis_name="s",
        num_cores=_NC, num_subcores=_NS)

    @functools.partial(
        pl.kernel,
        out_type=jax.ShapeDtypeStruct((b, df), jnp.float32),
        mesh=mesh,
        scratch_types=[
            pltpu.VMEM((bpw,), jnp.int32),
            pltpu.VMEM((2 * bpw,), jnp.int32),
            pltpu.VMEM((2 * bpw, 128), jnp.float32),
            pltpu.VMEM((bpw, tail.shape[1]), jnp.float32),
            pltpu.VMEM((bpw, df), jnp.float32),
            pltpu.VMEM((df,), jnp.int32),
            pltpu.SemaphoreType.DMA,
            pltpu.SemaphoreType.DMA,
        ],
        compiler_params=pltpu.CompilerParams(
            use_tc_tiling_on_sc=False, needs_layout_passes=False),
    )
    def body(lin_hbm, tail_hbm, idx_hbm, feat_hbm, fidx_hbm, out_hbm,
             idx_v, idx2_v, rows_v, tailrows_v, feat_v, fidx_v, sem, sem2):
        wid = lax.axis_index("s") * _NC + lax.axis_index("c")
        base = wid * bpw
        fbase = chunk * b + base
        pltpu.sync_copy(idx_hbm.at[pl.ds(base, bpw)], idx_v)
        cp2 = pltpu.async_copy(tail_hbm.at[idx_v], tailrows_v, sem2)
        lane = lax.iota(jnp.int32, _L)
        for g in range(bpw // _L):
            v = idx_v[pl.ds(g * _L, _L)]
            b2 = ((v >> 3) << 4) | (v & 7)
            pos = 2 * lane + (2 * g * _L)
            plsc.store_scatter(idx2_v, [pos], b2)
            plsc.store_scatter(idx2_v, [pos + 1], b2 + 8)
        cp = pltpu.async_copy(lin_hbm.at[idx2_v], rows_v, sem)
        pltpu.sync_copy(feat_hbm.at[pl.ds(fbase, bpw)], feat_v)
        pltpu.sync_copy(fidx_hbm, fidx_v)
        cp2.wait()
        cp.wait()
        csel, clane, ctail, cmask = [], [], [], []
        for c in range(nch):
            cols = fidx_v[pl.ds(c * _L, _L)]
            csel.append(jnp.minimum(cols >> 7, 1))
            clane.append(cols & 127)
            ctail.append(jnp.clip(cols - 2 * 128, 0, tail_w - 1))
            cmask.append(cols >= 2 * 128)

        def row_body(r, carry):
            r2 = jnp.full((_L,), 2 * r, jnp.int32)
            rs = jnp.full((_L,), r, jnp.int32)
            for c in range(nch):
                va = plsc.load_gather(rows_v, [r2 + csel[c], clane[c]])
                vb = plsc.load_gather(tailrows_v, [rs, ctail[c]])
                vals = jnp.where(cmask[c], vb, va)
                feat_v[r, pl.ds(c * _L, _L)] = feat_v[r, pl.ds(c * _L, _L)] - vals
            return carry

        lax.fori_loop(0, bpw, row_body, 0)
        pltpu.sync_copy(feat_v, out_hbm.at[pl.ds(base, bpw)])

    return body(lin, tail, idx, feat, fidx)


_NCHUNKS = 4


def kernel(predicted_key, features, lookup_table, lookup_key_indices,
           feature_indices):
    b = predicted_key.shape[0]
    key_table = jnp.take(lookup_table, lookup_key_indices, axis=1)
    t2 = jnp.sum(key_table * key_table, axis=1)[None, :]
    q2 = jnp.sum(predicted_key * predicted_key, axis=1, keepdims=True)
    ktT = key_table.T
    lin = _sc_linearize(lookup_table)
    tail = lax.slice(lookup_table, (0, 256), lookup_table.shape)
    bc = b // _NCHUNKS
    outs = []
    for c in range(_NCHUNKS):
        idx_c = _nearest_idx(predicted_key, ktT, t2, q2, c, bc)
        outs.append(_sc_residual(lin, tail, idx_c, features,
                                 feature_indices, c))
    return jnp.concatenate(outs, axis=0)
